# baseline (speedup 1.0000x reference)
import functools

import jax
import jax.numpy as jnp
from jax import lax
from jax.experimental import pallas as pl
from jax.experimental.pallas import tpu as pltpu
from jax.experimental.pallas import tpu_sc as plsc

N = 10000
D = 128
E = 320000
NT = 32
C = 128
EPT = 10240
K_IT = EPT // C
PAD = NT * EPT - E
N_ACC = 10112
RPT = N_ACC // 16

_mesh = plsc.VectorSubcoreMesh(core_axis_name="c", subcore_axis_name="s")


@functools.partial(
    pl.kernel,
    mesh=_mesh,
    out_type=jax.ShapeDtypeStruct((2, N_ACC, D), jnp.float32),
    scratch_types=[
        pltpu.VMEM((K_IT, C), jnp.int32),
        pltpu.VMEM((C, D), jnp.float32),
        pltpu.VMEM_SHARED((N_ACC, D), jnp.float32),
    ],
)
def _deg_sc(dst_hbm, ones_hbm, zeros_hbm, out_hbm, dst_v, ones_v, acc):
    c = lax.axis_index("c")
    s = lax.axis_index("s")
    wid = c * 16 + s
    pltpu.sync_copy(dst_hbm.at[wid], dst_v)
    pltpu.sync_copy(ones_hbm, ones_v)
    pltpu.sync_copy(zeros_hbm, acc.at[pl.ds(s * RPT, RPT)])
    plsc.subcore_barrier()

    def body(j, carry):
        pltpu.sync_copy(ones_v, acc.at[dst_v.at[j]], add=True)
        return carry

    lax.fori_loop(0, K_IT, body, 0)
    plsc.subcore_barrier()
    pltpu.sync_copy(acc.at[pl.ds(s * RPT, RPT)],
                    out_hbm.at[c, pl.ds(s * RPT, RPT)])


@functools.partial(
    pl.kernel,
    mesh=_mesh,
    out_type=jax.ShapeDtypeStruct((2, N_ACC, D), jnp.float32),
    scratch_types=[
        pltpu.VMEM((K_IT + 8, C), jnp.int32),
        pltpu.VMEM((K_IT, C), jnp.int32),
        pltpu.VMEM((C, D), jnp.float32),
        pltpu.VMEM_SHARED((N_ACC, D), jnp.float32),
        pltpu.SemaphoreType.DMA,
    ],
)
def _agg_sc(g_hbm, src_hbm, dst_hbm, zeros_hbm, out_hbm,
            src_v, dst_v, rows_v, acc, sem):
    c = lax.axis_index("c")
    s = lax.axis_index("s")
    wid = c * 16 + s
    pltpu.sync_copy(src_hbm.at[wid], src_v)
    pltpu.sync_copy(dst_hbm.at[wid], dst_v)
    pltpu.sync_copy(zeros_hbm, acc.at[pl.ds(s * RPT, RPT)])
    plsc.subcore_barrier()

    def body(j, carry):
        pltpu.async_copy(g_hbm.at[src_v.at[j]], rows_v, sem).wait()
        pltpu.sync_copy(rows_v, acc.at[dst_v.at[j]], add=True)
        return carry

    lax.fori_loop(0, K_IT, body, 0)
    plsc.subcore_barrier()
    pltpu.sync_copy(acc.at[pl.ds(s * RPT, RPT)],
                    out_hbm.at[c, pl.ds(s * RPT, RPT)])


def _prep_body(degp_ref, h_ref, norm_ref, g0_ref):
    deg = degp_ref[0, :N, 0:1] + degp_ref[1, :N, 0:1]
    norm = 1.0 / jnp.sqrt(jnp.maximum(deg, 1.0))
    norm_ref[...] = norm
    g0_ref[...] = h_ref[...] * norm


_prep_tc = pl.pallas_call(
    _prep_body,
    out_shape=(jax.ShapeDtypeStruct((N, 1), jnp.float32),
               jax.ShapeDtypeStruct((N, D), jnp.float32)),
)


def _layer_math(parts_ref, norm_ref, h_ref, w_ref, b_ref, gm_ref, bt_ref,
                residual):
    norm = norm_ref[...]
    agg = (parts_ref[0, :N, :] + parts_ref[1, :N, :]) * norm
    h2 = jnp.dot(agg, w_ref[...], preferred_element_type=jnp.float32)
    h2 = h2 + b_ref[...]
    mean = jnp.mean(h2, axis=0, keepdims=True)
    var = jnp.mean((h2 - mean) ** 2, axis=0, keepdims=True)
    h2 = (h2 - mean) / jnp.sqrt(var + 1e-5) * gm_ref[...] + bt_ref[...]
    h2 = jnp.maximum(h2, 0.0)
    if residual:
        h2 = h_ref[...] + h2
    return h2, norm


def _layer_body(parts_ref, norm_ref, h_ref, w_ref, b_ref, gm_ref, bt_ref,
                h_out_ref, g_out_ref, *, residual):
    h2, norm = _layer_math(parts_ref, norm_ref, h_ref, w_ref, b_ref,
                           gm_ref, bt_ref, residual)
    h_out_ref[...] = h2
    g_out_ref[...] = h2 * norm


def _make_layer_tc(residual):
    return pl.pallas_call(
        functools.partial(_layer_body, residual=residual),
        out_shape=(jax.ShapeDtypeStruct((N, D), jnp.float32),
                   jax.ShapeDtypeStruct((N, D), jnp.float32)),
    )


_layer_tc_first = _make_layer_tc(False)
_layer_tc_mid = _make_layer_tc(True)


def _final_body(parts_ref, norm_ref, h_ref, w_ref, b_ref, gm_ref, bt_ref,
                mw0_ref, mb0_ref, mw1_ref, mb1_ref, mw2_ref, mb2_ref,
                y_ref):
    h2, _ = _layer_math(parts_ref, norm_ref, h_ref, w_ref, b_ref,
                        gm_ref, bt_ref, True)
    hg = jnp.mean(h2, axis=0, keepdims=True)
    y = jnp.dot(hg, mw0_ref[...], preferred_element_type=jnp.float32)
    y = jnp.maximum(y + mb0_ref[...], 0.0)
    y = jnp.dot(y, mw1_ref[...], preferred_element_type=jnp.float32)
    y = jnp.maximum(y + mb1_ref[...], 0.0)
    y = jnp.dot(y, mw2_ref[...], preferred_element_type=jnp.float32)
    y_ref[...] = y + mb2_ref[...]


_final_tc = pl.pallas_call(
    _final_body,
    out_shape=jax.ShapeDtypeStruct((1, 10), jnp.float32),
)


def kernel(h, edge_index, e, Ws, bs, gammas, betas,
           mW0, mb0, mW1, mb1, mW2, mb2):
    src = edge_index[0]
    dst = edge_index[1]
    src_p = jnp.concatenate(
        [src, jnp.zeros((PAD,), jnp.int32)]).reshape(NT, K_IT, C)
    src_p = jnp.concatenate([src_p, jnp.zeros((NT, 8, C), jnp.int32)], axis=1)
    trash = N + (jnp.arange(PAD, dtype=jnp.int32) % (N_ACC - N))
    dst_p = jnp.concatenate([dst, trash]).reshape(NT, K_IT, C)

    onesD = jnp.ones((C, D), jnp.float32)
    zerosD = jnp.zeros((RPT, D), jnp.float32)

    deg_parts = _deg_sc(dst_p, onesD, zerosD)
    norm, g = _prep_tc(deg_parts, h)

    for i in range(5):
        parts = _agg_sc(g, src_p, dst_p, zerosD)
        wi = Ws[i]
        bi = bs[i].reshape(1, D)
        gi = gammas[i].reshape(1, D)
        ti = betas[i].reshape(1, D)
        if i == 0:
            h, g = _layer_tc_first(parts, norm, h, wi, bi, gi, ti)
        elif i < 4:
            h, g = _layer_tc_mid(parts, norm, h, wi, bi, gi, ti)
        else:
            y = _final_tc(parts, norm, h, wi, bi, gi, ti,
                          mW0, mb0.reshape(1, -1),
                          mW1, mb1.reshape(1, -1),
                          mW2, mb2.reshape(1, -1))
    return y


# speedup vs baseline: 2.6937x; 2.6937x over previous
import functools

import jax
import jax.numpy as jnp
from jax import lax
from jax.experimental import pallas as pl
from jax.experimental.pallas import tpu as pltpu
from jax.experimental.pallas import tpu_sc as plsc

N = 10000
D = 128
E = 320000
NT = 32
C = 128
EPT = 10240
K_IT = EPT // C
PAD = NT * EPT - E
N_ACC = 10112
RPT = N_ACC // 16

_mesh = plsc.VectorSubcoreMesh(core_axis_name="c", subcore_axis_name="s")


@functools.partial(
    pl.kernel,
    mesh=_mesh,
    out_type=jax.ShapeDtypeStruct((2, N_ACC, D), jnp.float32),
    scratch_types=[
        pltpu.VMEM((K_IT, C), jnp.int32),
        pltpu.VMEM((C, D), jnp.float32),
        pltpu.VMEM_SHARED((N_ACC, D), jnp.float32),
    ],
)
def _deg_sc(dst_hbm, ones_hbm, zeros_hbm, out_hbm, dst_v, ones_v, acc):
    c = lax.axis_index("c")
    s = lax.axis_index("s")
    wid = c * 16 + s
    pltpu.sync_copy(dst_hbm.at[wid], dst_v)
    pltpu.sync_copy(ones_hbm, ones_v)
    pltpu.sync_copy(zeros_hbm, acc.at[pl.ds(s * RPT, RPT)])
    plsc.subcore_barrier()

    def body(j, carry):
        pltpu.sync_copy(ones_v, acc.at[dst_v.at[j]], add=True)
        return carry

    lax.fori_loop(0, K_IT, body, 0)
    plsc.subcore_barrier()
    pltpu.sync_copy(acc.at[pl.ds(s * RPT, RPT)],
                    out_hbm.at[c, pl.ds(s * RPT, RPT)])


@functools.partial(
    pl.kernel,
    mesh=_mesh,
    out_type=jax.ShapeDtypeStruct((2, N_ACC, D), jnp.float32),
    scratch_types=[
        pltpu.VMEM((K_IT + 8, C), jnp.int32),
        pltpu.VMEM((K_IT, C), jnp.int32),
        pltpu.VMEM((C, D), jnp.float32),
        pltpu.VMEM_SHARED((N_ACC, D), jnp.float32),
        pltpu.SemaphoreType.DMA,
    ],
)
def _agg_sc(g_hbm, src_hbm, dst_hbm, zeros_hbm, out_hbm,
            src_v, dst_v, rows_v, acc, sem):
    c = lax.axis_index("c")
    s = lax.axis_index("s")
    wid = c * 16 + s
    pltpu.sync_copy(src_hbm.at[wid], src_v)
    pltpu.sync_copy(dst_hbm.at[wid], dst_v)
    pltpu.sync_copy(zeros_hbm, acc.at[pl.ds(s * RPT, RPT)])
    plsc.subcore_barrier()

    def body(j, carry):
        pltpu.async_copy(g_hbm.at[src_v.at[j]], rows_v, sem).wait()
        pltpu.sync_copy(rows_v, acc.at[dst_v.at[j]], add=True)
        return carry

    lax.fori_loop(0, K_IT, body, 0)
    plsc.subcore_barrier()
    pltpu.sync_copy(acc.at[pl.ds(s * RPT, RPT)],
                    out_hbm.at[c, pl.ds(s * RPT, RPT)])


def _prep_body(degp_ref, h_ref, norm_ref, g0_ref):
    deg = degp_ref[0, :N, 0:1] + degp_ref[1, :N, 0:1]
    norm = 1.0 / jnp.sqrt(jnp.maximum(deg, 1.0))
    norm_ref[...] = norm
    g0_ref[...] = h_ref[...] * norm


_prep_tc = pl.pallas_call(
    _prep_body,
    out_shape=(jax.ShapeDtypeStruct((N, 1), jnp.float32),
               jax.ShapeDtypeStruct((N, D), jnp.float32)),
)


def _layer_math(parts_ref, norm_ref, h_ref, w_ref, b_ref, gm_ref, bt_ref,
                residual):
    norm = norm_ref[...]
    agg = (parts_ref[0, :N, :] + parts_ref[1, :N, :]) * norm
    h2 = jnp.dot(agg, w_ref[...], preferred_element_type=jnp.float32)
    h2 = h2 + b_ref[...]
    mean = jnp.mean(h2, axis=0, keepdims=True)
    var = jnp.mean((h2 - mean) ** 2, axis=0, keepdims=True)
    h2 = (h2 - mean) / jnp.sqrt(var + 1e-5) * gm_ref[...] + bt_ref[...]
    h2 = jnp.maximum(h2, 0.0)
    if residual:
        h2 = h_ref[...] + h2
    return h2, norm


def _layer_body(parts_ref, norm_ref, h_ref, w_ref, b_ref, gm_ref, bt_ref,
                h_out_ref, g_out_ref, *, residual):
    h2, norm = _layer_math(parts_ref, norm_ref, h_ref, w_ref, b_ref,
                           gm_ref, bt_ref, residual)
    h_out_ref[...] = h2
    g_out_ref[...] = h2 * norm


def _make_layer_tc(residual):
    return pl.pallas_call(
        functools.partial(_layer_body, residual=residual),
        out_shape=(jax.ShapeDtypeStruct((N, D), jnp.float32),
                   jax.ShapeDtypeStruct((N, D), jnp.float32)),
    )


_layer_tc_first = _make_layer_tc(False)
_layer_tc_mid = _make_layer_tc(True)


def _final_body(parts_ref, norm_ref, h_ref, w_ref, b_ref, gm_ref, bt_ref,
                mw0_ref, mb0_ref, mw1_ref, mb1_ref, mw2_ref, mb2_ref,
                y_ref):
    h2, _ = _layer_math(parts_ref, norm_ref, h_ref, w_ref, b_ref,
                        gm_ref, bt_ref, True)
    hg = jnp.mean(h2, axis=0, keepdims=True)
    y = jnp.dot(hg, mw0_ref[...], preferred_element_type=jnp.float32)
    y = jnp.maximum(y + mb0_ref[...], 0.0)
    y = jnp.dot(y, mw1_ref[...], preferred_element_type=jnp.float32)
    y = jnp.maximum(y + mb1_ref[...], 0.0)
    y = jnp.dot(y, mw2_ref[...], preferred_element_type=jnp.float32)
    y_ref[...] = y + mb2_ref[...]


_final_tc = pl.pallas_call(
    _final_body,
    out_shape=jax.ShapeDtypeStruct((1, 10), jnp.float32),
)


def kernel(h, edge_index, e, Ws, bs, gammas, betas,
           mW0, mb0, mW1, mb1, mW2, mb2):
    src = edge_index[0]
    dst = edge_index[1]
    pad_src = jnp.arange(PAD, dtype=jnp.int32) % N
    src_p = jnp.concatenate([src, pad_src]).reshape(NT, K_IT, C)
    src_p = jnp.concatenate([src_p, jnp.zeros((NT, 8, C), jnp.int32)], axis=1)
    trash = N + (jnp.arange(PAD, dtype=jnp.int32) % (N_ACC - N))
    dst_p = jnp.concatenate([dst, trash]).reshape(NT, K_IT, C)

    onesD = jnp.ones((C, D), jnp.float32)
    zerosD = jnp.zeros((RPT, D), jnp.float32)

    deg_parts = _deg_sc(dst_p, onesD, zerosD)
    norm, g = _prep_tc(deg_parts, h)

    for i in range(5):
        parts = _agg_sc(g, src_p, dst_p, zerosD)
        wi = Ws[i]
        bi = bs[i].reshape(1, D)
        gi = gammas[i].reshape(1, D)
        ti = betas[i].reshape(1, D)
        if i == 0:
            h, g = _layer_tc_first(parts, norm, h, wi, bi, gi, ti)
        elif i < 4:
            h, g = _layer_tc_mid(parts, norm, h, wi, bi, gi, ti)
        else:
            y = _final_tc(parts, norm, h, wi, bi, gi, ti,
                          mW0, mb0.reshape(1, -1),
                          mW1, mb1.reshape(1, -1),
                          mW2, mb2.reshape(1, -1))
    return y


# speedup vs baseline: 3.3808x; 1.2551x over previous
import functools

import jax
import jax.numpy as jnp
from jax import lax
from jax.experimental import pallas as pl
from jax.experimental.pallas import tpu as pltpu
from jax.experimental.pallas import tpu_sc as plsc

N = 10000
D = 128
E = 320000
NT = 32
C = 128
EPT = 10240
K_IT = EPT // C
PAD = NT * EPT - E
N_ACC = 10112
RPT = N_ACC // 16

_mesh = plsc.VectorSubcoreMesh(core_axis_name="c", subcore_axis_name="s")


@functools.partial(
    pl.kernel,
    mesh=_mesh,
    out_type=jax.ShapeDtypeStruct((2, N_ACC, D), jnp.float32),
    scratch_types=[
        pltpu.VMEM((K_IT, C), jnp.int32),
        pltpu.VMEM((C, D), jnp.float32),
        pltpu.VMEM_SHARED((N_ACC, D), jnp.float32),
    ],
)
def _deg_sc(dst_hbm, ones_hbm, zeros_hbm, out_hbm, dst_v, ones_v, acc):
    c = lax.axis_index("c")
    s = lax.axis_index("s")
    wid = c * 16 + s
    pltpu.sync_copy(dst_hbm.at[wid], dst_v)
    pltpu.sync_copy(ones_hbm, ones_v)
    pltpu.sync_copy(zeros_hbm, acc.at[pl.ds(s * RPT, RPT)])
    plsc.subcore_barrier()

    def body(j, carry):
        pltpu.sync_copy(ones_v, acc.at[dst_v.at[j]], add=True)
        return carry

    lax.fori_loop(0, K_IT, body, 0)
    plsc.subcore_barrier()
    pltpu.sync_copy(acc.at[pl.ds(s * RPT, RPT)],
                    out_hbm.at[c, pl.ds(s * RPT, RPT)])


@functools.partial(
    pl.kernel,
    mesh=_mesh,
    out_type=jax.ShapeDtypeStruct((2, N_ACC, D), jnp.float32),
    scratch_types=[
        pltpu.VMEM((K_IT // 2, C), jnp.int32),
        pltpu.VMEM((K_IT // 2, C), jnp.int32),
        pltpu.VMEM((C, D), jnp.float32),
        pltpu.VMEM((C, D), jnp.float32),
        pltpu.VMEM_SHARED((N_ACC, D), jnp.float32),
        pltpu.SemaphoreType.DMA,
        pltpu.SemaphoreType.DMA,
        pltpu.SemaphoreType.DMA,
    ],
)
def _agg_sc(g_hbm, src_hbm, dst_hbm, zeros_hbm, out_hbm,
            src_v, dst_v, rows_a, rows_b, acc, sem_g, sem_a, sem_b):
    c = lax.axis_index("c")
    s = lax.axis_index("s")
    wid = c * 16 + s
    half = K_IT // 2
    pltpu.sync_copy(zeros_hbm, acc.at[pl.ds(s * RPT, RPT)])
    plsc.subcore_barrier()

    for p in range(2):
        pltpu.sync_copy(src_hbm.at[wid, pl.ds(p * half, half)], src_v)
        pltpu.sync_copy(dst_hbm.at[wid, pl.ds(p * half, half)], dst_v)
        pltpu.async_copy(g_hbm.at[src_v.at[0]], rows_a, sem_g).wait()
        pltpu.async_copy(rows_a, acc.at[dst_v.at[0]], sem_a, add=True)

        def body(i, carry):
            j0 = 2 * i
            j1 = j0 + 1
            j2 = j0 + 2
            pltpu.async_copy(g_hbm.at[src_v.at[j1]], rows_b, sem_g).wait()
            pltpu.make_async_copy(rows_a, acc.at[dst_v.at[j0]], sem_a).wait()
            pltpu.async_copy(rows_b, acc.at[dst_v.at[j1]], sem_b, add=True)
            pltpu.async_copy(g_hbm.at[src_v.at[j2]], rows_a, sem_g).wait()
            pltpu.make_async_copy(rows_b, acc.at[dst_v.at[j1]], sem_b).wait()
            pltpu.async_copy(rows_a, acc.at[dst_v.at[j2]], sem_a, add=True)
            return carry

        lax.fori_loop(0, half // 2 - 1, body, 0)
        pltpu.async_copy(g_hbm.at[src_v.at[half - 1]], rows_b, sem_g).wait()
        pltpu.make_async_copy(rows_a, acc.at[dst_v.at[half - 2]], sem_a).wait()
        pltpu.sync_copy(rows_b, acc.at[dst_v.at[half - 1]], add=True)
    plsc.subcore_barrier()
    pltpu.sync_copy(acc.at[pl.ds(s * RPT, RPT)],
                    out_hbm.at[c, pl.ds(s * RPT, RPT)])


def _prep_body(degp_ref, h_ref, norm_ref, g0_ref):
    deg = degp_ref[0, :N, 0:1] + degp_ref[1, :N, 0:1]
    norm = 1.0 / jnp.sqrt(jnp.maximum(deg, 1.0))
    norm_ref[...] = norm
    g0_ref[...] = h_ref[...] * norm


_prep_tc = pl.pallas_call(
    _prep_body,
    out_shape=(jax.ShapeDtypeStruct((N, 1), jnp.float32),
               jax.ShapeDtypeStruct((N, D), jnp.float32)),
)


def _layer_math(parts_ref, norm_ref, h_ref, w_ref, b_ref, gm_ref, bt_ref,
                residual):
    norm = norm_ref[...]
    agg = (parts_ref[0, :N, :] + parts_ref[1, :N, :]) * norm
    h2 = jnp.dot(agg, w_ref[...], preferred_element_type=jnp.float32)
    h2 = h2 + b_ref[...]
    mean = jnp.mean(h2, axis=0, keepdims=True)
    var = jnp.mean((h2 - mean) ** 2, axis=0, keepdims=True)
    h2 = (h2 - mean) / jnp.sqrt(var + 1e-5) * gm_ref[...] + bt_ref[...]
    h2 = jnp.maximum(h2, 0.0)
    if residual:
        h2 = h_ref[...] + h2
    return h2, norm


def _layer_body(parts_ref, norm_ref, h_ref, w_ref, b_ref, gm_ref, bt_ref,
                h_out_ref, g_out_ref, *, residual):
    h2, norm = _layer_math(parts_ref, norm_ref, h_ref, w_ref, b_ref,
                           gm_ref, bt_ref, residual)
    h_out_ref[...] = h2
    g_out_ref[...] = h2 * norm


def _make_layer_tc(residual):
    return pl.pallas_call(
        functools.partial(_layer_body, residual=residual),
        out_shape=(jax.ShapeDtypeStruct((N, D), jnp.float32),
                   jax.ShapeDtypeStruct((N, D), jnp.float32)),
    )


_layer_tc_first = _make_layer_tc(False)
_layer_tc_mid = _make_layer_tc(True)


def _final_body(parts_ref, norm_ref, h_ref, w_ref, b_ref, gm_ref, bt_ref,
                mw0_ref, mb0_ref, mw1_ref, mb1_ref, mw2_ref, mb2_ref,
                y_ref):
    h2, _ = _layer_math(parts_ref, norm_ref, h_ref, w_ref, b_ref,
                        gm_ref, bt_ref, True)
    hg = jnp.mean(h2, axis=0, keepdims=True)
    y = jnp.dot(hg, mw0_ref[...], preferred_element_type=jnp.float32)
    y = jnp.maximum(y + mb0_ref[...], 0.0)
    y = jnp.dot(y, mw1_ref[...], preferred_element_type=jnp.float32)
    y = jnp.maximum(y + mb1_ref[...], 0.0)
    y = jnp.dot(y, mw2_ref[...], preferred_element_type=jnp.float32)
    y_ref[...] = y + mb2_ref[...]


_final_tc = pl.pallas_call(
    _final_body,
    out_shape=jax.ShapeDtypeStruct((1, 10), jnp.float32),
)


def kernel(h, edge_index, e, Ws, bs, gammas, betas,
           mW0, mb0, mW1, mb1, mW2, mb2):
    src = edge_index[0]
    dst = edge_index[1]
    pad_src = jnp.arange(PAD, dtype=jnp.int32) % N
    src_p = jnp.concatenate([src, pad_src]).reshape(NT, K_IT, C)
    trash = N + (jnp.arange(PAD, dtype=jnp.int32) % (N_ACC - N))
    dst_p = jnp.concatenate([dst, trash]).reshape(NT, K_IT, C)

    onesD = jnp.ones((C, D), jnp.float32)
    zerosD = jnp.zeros((RPT, D), jnp.float32)

    deg_parts = _deg_sc(dst_p, onesD, zerosD)
    norm, g = _prep_tc(deg_parts, h)

    for i in range(5):
        parts = _agg_sc(g, src_p, dst_p, zerosD)
        wi = Ws[i]
        bi = bs[i].reshape(1, D)
        gi = gammas[i].reshape(1, D)
        ti = betas[i].reshape(1, D)
        if i == 0:
            h, g = _layer_tc_first(parts, norm, h, wi, bi, gi, ti)
        elif i < 4:
            h, g = _layer_tc_mid(parts, norm, h, wi, bi, gi, ti)
        else:
            y = _final_tc(parts, norm, h, wi, bi, gi, ti,
                          mW0, mb0.reshape(1, -1),
                          mW1, mb1.reshape(1, -1),
                          mW2, mb2.reshape(1, -1))
    return y
